# baseline (speedup 1.0000x reference)
import jax
import jax.numpy as jnp
from jax import lax
from jax.experimental import pallas as pl
from jax.experimental.pallas import tpu as pltpu
from jax.experimental.pallas import tpu_sc as plsc

BATCH = 16384
DIM = 128
NSITES = 16
LANES = 16
NC, NS = 2, 16
NW = NC * NS
B_SC = 8192
B_TC = BATCH - B_SC
BPW = B_SC // NW
CH = 128
NCHUNK = BPW // CH
NBUF = 2
TC_BLK = 1024


def _sc_body(in_hbm, sites_hbm, masks_hbm, idx_hbm, out_hbm,
             idx_v, sites_v, masks_v, in_buf, out_buf, sem_in, sem_out):
    wid = lax.axis_index("s") * NC + lax.axis_index("c")
    base = wid * BPW

    pltpu.sync_copy(idx_hbm.at[pl.ds(base, BPW)], idx_v)
    pltpu.sync_copy(sites_hbm, sites_v)
    pltpu.sync_copy(masks_hbm, masks_v)

    in_copies = [None] * NCHUNK
    out_copies = [None] * NCHUNK
    in_copies[0] = pltpu.async_copy(
        in_hbm.at[pl.ds(base, CH)], in_buf.at[0], sem_in.at[0])

    for g in range(NCHUNK):
        slot = g % NBUF
        if g + 1 < NCHUNK:
            nslot = (g + 1) % NBUF
            in_copies[g + 1] = pltpu.async_copy(
                in_hbm.at[pl.ds(base + (g + 1) * CH, CH)],
                in_buf.at[nslot], sem_in.at[nslot])
        in_copies[g].wait()
        if g >= NBUF:
            out_copies[g - NBUF].wait()

        zeros16 = jnp.zeros((LANES,), dtype=jnp.int32)

        @plsc.parallel_loop(0, CH, unroll=4)
        def _rows(r, slot=slot, off=g * CH):
            jvec = plsc.load_gather(idx_v, [zeros16 + (off + r)])
            mi = plsc.load_gather(masks_v, [jvec])
            si = plsc.load_gather(sites_v, [jvec])
            for q in range(DIM // LANES):
                v = in_buf[slot, r, pl.ds(q * LANES, LANES)]
                out_buf[slot, r, pl.ds(q * LANES, LANES)] = v * mi + si

        out_copies[g] = pltpu.async_copy(
            out_buf.at[slot], out_hbm.at[pl.ds(base + g * CH, CH)],
            sem_out.at[slot])

    for g in range(max(0, NCHUNK - NBUF), NCHUNK):
        out_copies[g].wait()


def _sc_call(inputs, sites, masks, idx):
    mesh = plsc.VectorSubcoreMesh(core_axis_name="c", subcore_axis_name="s")
    return pl.kernel(
        _sc_body,
        out_type=jax.ShapeDtypeStruct((B_SC, DIM), jnp.float32),
        mesh=mesh,
        compiler_params=pltpu.CompilerParams(needs_layout_passes=False),
        scratch_types=[
            pltpu.VMEM((BPW,), jnp.int32),
            pltpu.VMEM((NSITES,), jnp.float32),
            pltpu.VMEM((NSITES,), jnp.float32),
            pltpu.VMEM((NBUF, CH, DIM), jnp.float32),
            pltpu.VMEM((NBUF, CH, DIM), jnp.float32),
            pltpu.SemaphoreType.DMA((NBUF,)),
            pltpu.SemaphoreType.DMA((NBUF,)),
        ],
    )(inputs, sites, masks, idx)


def _tc_body(idx_ref, sites_ref, masks_ref, in_ref, out_ref):
    idx = idx_ref[...]
    sel = jax.lax.broadcasted_iota(jnp.int32, (TC_BLK, NSITES), 1)
    oh = (idx == sel).astype(jnp.float32)
    m = jnp.dot(oh, masks_ref[...],
                preferred_element_type=jnp.float32)
    s = jnp.dot(oh, sites_ref[...],
                preferred_element_type=jnp.float32)
    out_ref[...] = in_ref[...] * m + s


def _tc_call(inputs, sites2d, masks2d, idx2d):
    grid = (B_TC // TC_BLK,)
    return pl.pallas_call(
        _tc_body,
        grid=grid,
        in_specs=[
            pl.BlockSpec((TC_BLK, 1), lambda i: (i, 0)),
            pl.BlockSpec((NSITES, 1), lambda i: (0, 0)),
            pl.BlockSpec((NSITES, 1), lambda i: (0, 0)),
            pl.BlockSpec((TC_BLK, DIM), lambda i: (i, 0)),
        ],
        out_specs=pl.BlockSpec((TC_BLK, DIM), lambda i: (i, 0)),
        out_shape=jax.ShapeDtypeStruct((B_TC, DIM), jnp.float32),
    )(idx2d, sites2d, masks2d, inputs)


def kernel(inputs, injection_sites, masks, random_indexes):
    sites_f = injection_sites.reshape(NSITES).astype(jnp.float32)
    masks_f = masks.reshape(NSITES).astype(jnp.float32)
    idx32 = random_indexes.astype(jnp.int32)

    out_sc = _sc_call(inputs[:B_SC], sites_f, masks_f, idx32[:B_SC])
    out_tc = _tc_call(
        inputs[B_SC:],
        injection_sites.astype(jnp.float32),
        masks.astype(jnp.float32),
        idx32[B_SC:].reshape(B_TC, 1),
    )
    return jnp.concatenate([out_sc, out_tc], axis=0)


# speedup vs baseline: 1.4631x; 1.4631x over previous
import jax
import jax.numpy as jnp
from jax import lax
from jax.experimental import pallas as pl
from jax.experimental.pallas import tpu as pltpu
from jax.experimental.pallas import tpu_sc as plsc

BATCH = 16384
DIM = 128
NSITES = 16
LANES = 16
NC, NS = 2, 16
NW = NC * NS
BPW = BATCH // NW
CH = 128
NCHUNK = BPW // CH
NBUF = 2


def _body(in_hbm, sites_hbm, masks_hbm, idx_hbm, out_hbm,
          idx_v, sites_v, masks_v, in_buf, out_buf, sem_in, sem_out):
    wid = lax.axis_index("s") * NC + lax.axis_index("c")
    base = wid * BPW

    pltpu.sync_copy(idx_hbm.at[pl.ds(base, BPW)], idx_v)
    pltpu.sync_copy(sites_hbm, sites_v)
    pltpu.sync_copy(masks_hbm, masks_v)

    in_copies = [None] * NCHUNK
    out_copies = [None] * NCHUNK
    in_copies[0] = pltpu.async_copy(
        in_hbm.at[pl.ds(base, CH)], in_buf.at[0], sem_in.at[0])

    for g in range(NCHUNK):
        slot = g % NBUF
        if g + 1 < NCHUNK:
            nslot = (g + 1) % NBUF
            in_copies[g + 1] = pltpu.async_copy(
                in_hbm.at[pl.ds(base + (g + 1) * CH, CH)],
                in_buf.at[nslot], sem_in.at[nslot])
        in_copies[g].wait()
        if g >= NBUF:
            out_copies[g - NBUF].wait()

        zeros16 = jnp.zeros((LANES,), dtype=jnp.int32)

        @plsc.parallel_loop(0, CH, unroll=4)
        def _rows(r, slot=slot, off=g * CH):
            jvec = plsc.load_gather(idx_v, [zeros16 + (off + r)])
            mi = plsc.load_gather(masks_v, [jvec, zeros16])
            si = plsc.load_gather(sites_v, [jvec, zeros16])
            for q in range(DIM // LANES):
                v = in_buf[slot, r, pl.ds(q * LANES, LANES)]
                out_buf[slot, r, pl.ds(q * LANES, LANES)] = v * mi + si

        out_copies[g] = pltpu.async_copy(
            out_buf.at[slot], out_hbm.at[pl.ds(base + g * CH, CH)],
            sem_out.at[slot])

    for g in range(max(0, NCHUNK - NBUF), NCHUNK):
        out_copies[g].wait()


def kernel(inputs, injection_sites, masks, random_indexes):
    mesh = plsc.VectorSubcoreMesh(core_axis_name="c", subcore_axis_name="s")
    return pl.kernel(
        _body,
        out_type=jax.ShapeDtypeStruct((BATCH, DIM), jnp.float32),
        mesh=mesh,
        compiler_params=pltpu.CompilerParams(needs_layout_passes=False),
        scratch_types=[
            pltpu.VMEM((BPW,), jnp.int32),
            pltpu.VMEM((NSITES, 1), jnp.float32),
            pltpu.VMEM((NSITES, 1), jnp.float32),
            pltpu.VMEM((NBUF, CH, DIM), jnp.float32),
            pltpu.VMEM((NBUF, CH, DIM), jnp.float32),
            pltpu.SemaphoreType.DMA((NBUF,)),
            pltpu.SemaphoreType.DMA((NBUF,)),
        ],
    )(inputs, injection_sites, masks, random_indexes)
